# baseline (speedup 1.0000x reference)
import functools

import jax
import jax.numpy as jnp
from jax import lax
from jax.experimental import pallas as pl
from jax.experimental.pallas import tpu as pltpu
from jax.experimental.pallas import tpu_sc as plsc

N = 10000
E = 320000
D_IN = 128
D_H = 128
D_OUT = 64

NC = 2
NS = 16
NW = NC * NS
CH = 128

NCHA = -(-(E // NW) // CH)
EPTA = NCHA * CH

NCHD = -(-(E // NS) // CH)
EPTD = NCHD * CH

NP = N + 128
WR = 632
WR_LAST = N - (NS - 1) * WR
DEG_W = 16

_MESH = plsc.VectorSubcoreMesh(
    core_axis_name="c", subcore_axis_name="s", num_cores=NC, num_subcores=NS
)
_PARAMS = pltpu.CompilerParams(use_tc_tiling_on_sc=False)


@functools.partial(
    pl.kernel,
    out_type=(
        jax.ShapeDtypeStruct((N, DEG_W), jnp.float32),
        jax.ShapeDtypeStruct((N, DEG_W), jnp.float32),
    ),
    mesh=_MESH,
    compiler_params=_PARAMS,
    scratch_types=[
        pltpu.VMEM_SHARED((NP, DEG_W), jnp.float32),
        pltpu.VMEM((CH,), jnp.int32),
        pltpu.VMEM((CH,), jnp.int32),
        pltpu.VMEM((CH, DEG_W), jnp.float32),
        pltpu.SemaphoreType.DMA,
        pltpu.SemaphoreType.DMA,
    ],
)
def _deg_kernel(srcp, dstp, z16, ones, dout, din, acc, i0, i1, ones_v, semi0, semi1):
    c = lax.axis_index("c")
    s = lax.axis_index("s")
    ebase = s * EPTD
    pltpu.sync_copy(ones, ones_v)

    ibuf = (i0, i1)
    semi = (semi0, semi1)

    @pl.when(s == 0)
    def _():
        pltpu.sync_copy(z16, acc.at[pl.ds(0, N)])

    def run(arr, out):
        def load_idx(j, p):
            pltpu.async_copy(arr.at[pl.ds(ebase + j * CH, CH)], ibuf[p], semi[p])

        def wait_idx(j, p):
            pltpu.make_async_copy(
                arr.at[pl.ds(ebase + j * CH, CH)], ibuf[p], semi[p]
            ).wait()

        load_idx(0, 0)
        load_idx(1, 1)
        plsc.subcore_barrier()

        def body(j, _):
            for p in (0, 1):
                @pl.when(j % 2 == p)
                def _():
                    wait_idx(j, p)
                    pltpu.sync_copy(ones_v, acc.at[ibuf[p]], add=True)

                    @pl.when(j + 2 < NCHD)
                    def _():
                        load_idx(j + 2, p)

            return ()

        lax.fori_loop(0, NCHD, body, ())
        plsc.subcore_barrier()

        @pl.when(s < NS - 1)
        def _():
            sl = pl.ds(s * WR, WR)
            pltpu.sync_copy(acc.at[sl], out.at[sl])

        @pl.when(s == NS - 1)
        def _():
            sl = pl.ds(s * WR, WR_LAST)
            pltpu.sync_copy(acc.at[sl], out.at[sl])

    @pl.when(c == 0)
    def _():
        run(srcp, dout)

    @pl.when(c == 1)
    def _():
        run(dstp, din)


def _make_agg(D):
    @functools.partial(
        pl.kernel,
        out_type=jax.ShapeDtypeStruct((NC, N, D), jnp.float32),
        mesh=_MESH,
        compiler_params=_PARAMS,
        scratch_types=[
            pltpu.VMEM_SHARED((NP, D), jnp.float32),
            pltpu.VMEM((CH,), jnp.int32),
            pltpu.VMEM((CH,), jnp.int32),
            pltpu.VMEM((CH,), jnp.int32),
            pltpu.VMEM((CH,), jnp.int32),
            pltpu.VMEM((CH, D), jnp.float32),
            pltpu.VMEM((CH, D), jnp.float32),
            pltpu.SemaphoreType.DMA,
            pltpu.SemaphoreType.DMA,
            pltpu.SemaphoreType.DMA,
            pltpu.SemaphoreType.DMA,
        ],
    )
    def agg(h, srcp, dstp, zd, out, acc,
            is0, is1, id0, id1, rows0, rows1, semi0, semi1, semg0, semg1):
        c = lax.axis_index("c")
        s = lax.axis_index("s")
        wid = c * NS + s
        ebase = wid * EPTA

        isbuf = (is0, is1)
        idbuf = (id0, id1)
        rows = (rows0, rows1)
        semi = (semi0, semi1)
        semg = (semg0, semg1)

        def load_idx(j, p):
            pltpu.async_copy(srcp.at[pl.ds(ebase + j * CH, CH)], isbuf[p], semi[p])
            pltpu.async_copy(dstp.at[pl.ds(ebase + j * CH, CH)], idbuf[p], semi[p])

        def wait_idx(j, p):
            pltpu.make_async_copy(
                srcp.at[pl.ds(ebase + j * CH, CH)], isbuf[p], semi[p]
            ).wait()
            pltpu.make_async_copy(
                dstp.at[pl.ds(ebase + j * CH, CH)], idbuf[p], semi[p]
            ).wait()

        def gather(p):
            pltpu.async_copy(h.at[isbuf[p]], rows[p], semg[p])

        def wait_gather(p):
            pltpu.make_async_copy(h.at[isbuf[p]], rows[p], semg[p]).wait()

        @pl.when(s == 0)
        def _():
            pltpu.sync_copy(zd, acc.at[pl.ds(0, N)])

        load_idx(0, 0)
        load_idx(1, 1)
        plsc.subcore_barrier()
        wait_idx(0, 0)
        gather(0)

        def body(j, _):
            for p in (0, 1):
                @pl.when(j % 2 == p)
                def _():
                    q = 1 - p

                    @pl.when(j + 1 < NCHA)
                    def _():
                        wait_idx(j + 1, q)
                        gather(q)

                    wait_gather(p)
                    pltpu.sync_copy(rows[p], acc.at[idbuf[p]], add=True)

                    @pl.when(j + 2 < NCHA)
                    def _():
                        load_idx(j + 2, p)

            return ()

        lax.fori_loop(0, NCHA, body, ())
        plsc.subcore_barrier()

        @pl.when(s < NS - 1)
        def _():
            sl = pl.ds(s * WR, WR)
            pltpu.sync_copy(acc.at[sl], out.at[c, sl])

        @pl.when(s == NS - 1)
        def _():
            sl = pl.ds(s * WR, WR_LAST)
            pltpu.sync_copy(acc.at[sl], out.at[c, sl])

    return agg


_agg128 = _make_agg(D_H)
_agg64 = _make_agg(D_OUT)


_BLK = 1000


def _norm_from(dp):
    return lax.rsqrt(jnp.maximum(dp[:, 0], 1.0))


def _matmul(x, w):
    m, k = x.shape
    d = w.shape[1]

    def body(x_ref, w_ref, o_ref):
        o_ref[...] = jnp.dot(x_ref[...], w_ref[...], preferred_element_type=jnp.float32)

    return pl.pallas_call(
        body,
        grid=(m // _BLK,),
        in_specs=[
            pl.BlockSpec((_BLK, k), lambda i: (i, 0)),
            pl.BlockSpec((k, d), lambda i: (0, 0)),
        ],
        out_specs=pl.BlockSpec((_BLK, d), lambda i: (i, 0)),
        out_shape=jax.ShapeDtypeStruct((m, d), jnp.float32),
    )(x, w)


def _scale(u, deg_out):
    m, d = u.shape

    def body(u_ref, dp_ref, o_ref):
        o_ref[...] = u_ref[...] * _norm_from(dp_ref[...])[:, None]

    return pl.pallas_call(
        body,
        grid=(m // _BLK,),
        in_specs=[
            pl.BlockSpec((_BLK, d), lambda i: (i, 0)),
            pl.BlockSpec((_BLK, DEG_W), lambda i: (i, 0)),
        ],
        out_specs=pl.BlockSpec((_BLK, d), lambda i: (i, 0)),
        out_shape=jax.ShapeDtypeStruct((m, d), jnp.float32),
    )(u, deg_out)


def _layer2_in(aggp, deg_in, deg_out, b1r, w2):
    d = w2.shape[1]

    def body(ap_ref, di_ref, do_ref, b_ref, w_ref, o_ref):
        ap = ap_ref[...]
        agg = ap[0] + ap[1]
        z = agg * _norm_from(di_ref[...])[:, None] + b_ref[...]
        z = jnp.maximum(z, 0.0)
        z = z * _norm_from(do_ref[...])[:, None]
        o_ref[...] = jnp.dot(z, w_ref[...], preferred_element_type=jnp.float32)

    return pl.pallas_call(
        body,
        grid=(N // _BLK,),
        in_specs=[
            pl.BlockSpec((NC, _BLK, D_H), lambda i: (0, i, 0)),
            pl.BlockSpec((_BLK, DEG_W), lambda i: (i, 0)),
            pl.BlockSpec((_BLK, DEG_W), lambda i: (i, 0)),
            pl.BlockSpec((1, D_H), lambda i: (0, 0)),
            pl.BlockSpec((D_H, d), lambda i: (0, 0)),
        ],
        out_specs=pl.BlockSpec((_BLK, d), lambda i: (i, 0)),
        out_shape=jax.ShapeDtypeStruct((N, d), jnp.float32),
    )(aggp, deg_in, deg_out, b1r, w2)


def _final(aggp, deg_in, b2r):
    def body(ap_ref, di_ref, b_ref, o_ref):
        ap = ap_ref[...]
        agg = ap[0] + ap[1]
        o_ref[...] = agg * _norm_from(di_ref[...])[:, None] + b_ref[...]

    return pl.pallas_call(
        body,
        grid=(N // _BLK,),
        in_specs=[
            pl.BlockSpec((NC, _BLK, D_OUT), lambda i: (0, i, 0)),
            pl.BlockSpec((_BLK, DEG_W), lambda i: (i, 0)),
            pl.BlockSpec((1, D_OUT), lambda i: (0, 0)),
        ],
        out_specs=pl.BlockSpec((_BLK, D_OUT), lambda i: (i, 0)),
        out_shape=jax.ShapeDtypeStruct((N, D_OUT), jnp.float32),
    )(aggp, deg_in, b2r)


def _pad_tiles(arr, n_tiles, padded_len, dump):
    per = E // n_tiles
    pad = padded_len - per
    a2 = arr.reshape(n_tiles, per)
    if dump:
        fill = N + (jnp.arange(pad, dtype=jnp.int32) % (NP - N))
        block = jnp.broadcast_to(fill[None, :], (n_tiles, pad))
    else:
        block = jnp.zeros((n_tiles, pad), dtype=jnp.int32)
    return jnp.concatenate([a2, block], axis=1).reshape(-1)


def kernel(in_feat, edge_index, W1, b1, W2, b2):
    src, dst = edge_index[0], edge_index[1]
    src_a = _pad_tiles(src, NW, EPTA, dump=False)
    dst_a = _pad_tiles(dst, NW, EPTA, dump=True)
    src_d = _pad_tiles(src, NS, EPTD, dump=True)
    dst_d = _pad_tiles(dst, NS, EPTD, dump=True)

    z16 = jnp.zeros((N, DEG_W), jnp.float32)
    ones = jnp.ones((CH, DEG_W), jnp.float32)
    z128 = jnp.zeros((N, D_H), jnp.float32)
    z64 = jnp.zeros((N, D_OUT), jnp.float32)

    deg_out, deg_in = _deg_kernel(src_d, dst_d, z16, ones)
    u = _matmul(in_feat, W1)
    h1 = _scale(u, deg_out)
    agg1 = _agg128(h1, src_a, dst_a, z128)
    h2 = _layer2_in(agg1, deg_in, deg_out, b1.reshape(1, D_H), W2)
    agg2 = _agg64(h2, src_a, dst_a, z64)
    return _final(agg2, deg_in, b2.reshape(1, D_OUT))


# speedup vs baseline: 1.0146x; 1.0146x over previous
import functools

import jax
import jax.numpy as jnp
from jax import lax
from jax.experimental import pallas as pl
from jax.experimental.pallas import tpu as pltpu
from jax.experimental.pallas import tpu_sc as plsc

N = 10000
E = 320000
D_IN = 128
D_H = 128
D_OUT = 64

NC = 2
NS = 16
NW = NC * NS
CH = 64

NCHA = -(-(E // NW) // CH)
EPTA = NCHA * CH

NCHD = -(-(E // NS) // CH)
EPTD = NCHD * CH

NP = N + 128
WR = 632
WR_LAST = N - (NS - 1) * WR
DEG_W = 16

_MESH = plsc.VectorSubcoreMesh(
    core_axis_name="c", subcore_axis_name="s", num_cores=NC, num_subcores=NS
)
_PARAMS = pltpu.CompilerParams(use_tc_tiling_on_sc=False)


@functools.partial(
    pl.kernel,
    out_type=(
        jax.ShapeDtypeStruct((N, DEG_W), jnp.float32),
        jax.ShapeDtypeStruct((N, DEG_W), jnp.float32),
    ),
    mesh=_MESH,
    compiler_params=_PARAMS,
    scratch_types=[
        pltpu.VMEM_SHARED((NP, DEG_W), jnp.float32),
        pltpu.VMEM((CH,), jnp.int32),
        pltpu.VMEM((CH,), jnp.int32),
        pltpu.VMEM((CH, DEG_W), jnp.float32),
        pltpu.SemaphoreType.DMA,
        pltpu.SemaphoreType.DMA,
    ],
)
def _deg_kernel(srcp, dstp, z16, ones, dout, din, acc, i0, i1, ones_v, semi0, semi1):
    c = lax.axis_index("c")
    s = lax.axis_index("s")
    ebase = s * EPTD
    pltpu.sync_copy(ones, ones_v)

    ibuf = (i0, i1)
    semi = (semi0, semi1)

    @pl.when(s == 0)
    def _():
        pltpu.sync_copy(z16, acc.at[pl.ds(0, N)])

    def run(arr, out):
        def load_idx(j, p):
            pltpu.async_copy(arr.at[pl.ds(ebase + j * CH, CH)], ibuf[p], semi[p])

        def wait_idx(j, p):
            pltpu.make_async_copy(
                arr.at[pl.ds(ebase + j * CH, CH)], ibuf[p], semi[p]
            ).wait()

        load_idx(0, 0)
        load_idx(1, 1)
        plsc.subcore_barrier()

        def body(j, _):
            for p in (0, 1):
                @pl.when(j % 2 == p)
                def _():
                    wait_idx(j, p)
                    pltpu.sync_copy(ones_v, acc.at[ibuf[p]], add=True)

                    @pl.when(j + 2 < NCHD)
                    def _():
                        load_idx(j + 2, p)

            return ()

        lax.fori_loop(0, NCHD, body, ())
        plsc.subcore_barrier()

        @pl.when(s < NS - 1)
        def _():
            sl = pl.ds(s * WR, WR)
            pltpu.sync_copy(acc.at[sl], out.at[sl])

        @pl.when(s == NS - 1)
        def _():
            sl = pl.ds(s * WR, WR_LAST)
            pltpu.sync_copy(acc.at[sl], out.at[sl])

    @pl.when(c == 0)
    def _():
        run(srcp, dout)

    @pl.when(c == 1)
    def _():
        run(dstp, din)


def _make_agg(D):
    @functools.partial(
        pl.kernel,
        out_type=jax.ShapeDtypeStruct((NC, N, D), jnp.float32),
        mesh=_MESH,
        compiler_params=_PARAMS,
        scratch_types=[
            pltpu.VMEM_SHARED((NP, D), jnp.float32),
            pltpu.VMEM((CH,), jnp.int32),
            pltpu.VMEM((CH,), jnp.int32),
            pltpu.VMEM((CH,), jnp.int32),
            pltpu.VMEM((CH,), jnp.int32),
            pltpu.VMEM((CH, D), jnp.float32),
            pltpu.VMEM((CH, D), jnp.float32),
            pltpu.SemaphoreType.DMA,
            pltpu.SemaphoreType.DMA,
            pltpu.SemaphoreType.DMA,
            pltpu.SemaphoreType.DMA,
        ],
    )
    def agg(h, srcp, dstp, zd, out, acc,
            is0, is1, id0, id1, rows0, rows1, semi0, semi1, semg0, semg1):
        c = lax.axis_index("c")
        s = lax.axis_index("s")
        wid = c * NS + s
        ebase = wid * EPTA

        isbuf = (is0, is1)
        idbuf = (id0, id1)
        rows = (rows0, rows1)
        semi = (semi0, semi1)
        semg = (semg0, semg1)

        def load_idx(j, p):
            pltpu.async_copy(srcp.at[pl.ds(ebase + j * CH, CH)], isbuf[p], semi[p])
            pltpu.async_copy(dstp.at[pl.ds(ebase + j * CH, CH)], idbuf[p], semi[p])

        def wait_idx(j, p):
            pltpu.make_async_copy(
                srcp.at[pl.ds(ebase + j * CH, CH)], isbuf[p], semi[p]
            ).wait()
            pltpu.make_async_copy(
                dstp.at[pl.ds(ebase + j * CH, CH)], idbuf[p], semi[p]
            ).wait()

        def gather(p):
            pltpu.async_copy(h.at[isbuf[p]], rows[p], semg[p])

        def wait_gather(p):
            pltpu.make_async_copy(h.at[isbuf[p]], rows[p], semg[p]).wait()

        @pl.when(s == 0)
        def _():
            pltpu.sync_copy(zd, acc.at[pl.ds(0, N)])

        load_idx(0, 0)
        load_idx(1, 1)
        plsc.subcore_barrier()
        wait_idx(0, 0)
        gather(0)

        def body(j, _):
            for p in (0, 1):
                @pl.when(j % 2 == p)
                def _():
                    q = 1 - p

                    @pl.when(j + 1 < NCHA)
                    def _():
                        wait_idx(j + 1, q)
                        gather(q)

                    wait_gather(p)
                    pltpu.sync_copy(rows[p], acc.at[idbuf[p]], add=True)

                    @pl.when(j + 2 < NCHA)
                    def _():
                        load_idx(j + 2, p)

            return ()

        lax.fori_loop(0, NCHA, body, ())
        plsc.subcore_barrier()

        @pl.when(s < NS - 1)
        def _():
            sl = pl.ds(s * WR, WR)
            pltpu.sync_copy(acc.at[sl], out.at[c, sl])

        @pl.when(s == NS - 1)
        def _():
            sl = pl.ds(s * WR, WR_LAST)
            pltpu.sync_copy(acc.at[sl], out.at[c, sl])

    return agg


_agg128 = _make_agg(D_H)
_agg64 = _make_agg(D_OUT)


_BLK = 1000


def _norm_from(dp):
    return lax.rsqrt(jnp.maximum(dp[:, 0], 1.0))


def _matmul(x, w):
    m, k = x.shape
    d = w.shape[1]

    def body(x_ref, w_ref, o_ref):
        o_ref[...] = jnp.dot(x_ref[...], w_ref[...], preferred_element_type=jnp.float32)

    return pl.pallas_call(
        body,
        grid=(m // _BLK,),
        in_specs=[
            pl.BlockSpec((_BLK, k), lambda i: (i, 0)),
            pl.BlockSpec((k, d), lambda i: (0, 0)),
        ],
        out_specs=pl.BlockSpec((_BLK, d), lambda i: (i, 0)),
        out_shape=jax.ShapeDtypeStruct((m, d), jnp.float32),
    )(x, w)


def _scale(u, deg_out):
    m, d = u.shape

    def body(u_ref, dp_ref, o_ref):
        o_ref[...] = u_ref[...] * _norm_from(dp_ref[...])[:, None]

    return pl.pallas_call(
        body,
        grid=(m // _BLK,),
        in_specs=[
            pl.BlockSpec((_BLK, d), lambda i: (i, 0)),
            pl.BlockSpec((_BLK, DEG_W), lambda i: (i, 0)),
        ],
        out_specs=pl.BlockSpec((_BLK, d), lambda i: (i, 0)),
        out_shape=jax.ShapeDtypeStruct((m, d), jnp.float32),
    )(u, deg_out)


def _layer2_in(aggp, deg_in, deg_out, b1r, w2):
    d = w2.shape[1]

    def body(ap_ref, di_ref, do_ref, b_ref, w_ref, o_ref):
        ap = ap_ref[...]
        agg = ap[0] + ap[1]
        z = agg * _norm_from(di_ref[...])[:, None] + b_ref[...]
        z = jnp.maximum(z, 0.0)
        z = z * _norm_from(do_ref[...])[:, None]
        o_ref[...] = jnp.dot(z, w_ref[...], preferred_element_type=jnp.float32)

    return pl.pallas_call(
        body,
        grid=(N // _BLK,),
        in_specs=[
            pl.BlockSpec((NC, _BLK, D_H), lambda i: (0, i, 0)),
            pl.BlockSpec((_BLK, DEG_W), lambda i: (i, 0)),
            pl.BlockSpec((_BLK, DEG_W), lambda i: (i, 0)),
            pl.BlockSpec((1, D_H), lambda i: (0, 0)),
            pl.BlockSpec((D_H, d), lambda i: (0, 0)),
        ],
        out_specs=pl.BlockSpec((_BLK, d), lambda i: (i, 0)),
        out_shape=jax.ShapeDtypeStruct((N, d), jnp.float32),
    )(aggp, deg_in, deg_out, b1r, w2)


def _final(aggp, deg_in, b2r):
    def body(ap_ref, di_ref, b_ref, o_ref):
        ap = ap_ref[...]
        agg = ap[0] + ap[1]
        o_ref[...] = agg * _norm_from(di_ref[...])[:, None] + b_ref[...]

    return pl.pallas_call(
        body,
        grid=(N // _BLK,),
        in_specs=[
            pl.BlockSpec((NC, _BLK, D_OUT), lambda i: (0, i, 0)),
            pl.BlockSpec((_BLK, DEG_W), lambda i: (i, 0)),
            pl.BlockSpec((1, D_OUT), lambda i: (0, 0)),
        ],
        out_specs=pl.BlockSpec((_BLK, D_OUT), lambda i: (i, 0)),
        out_shape=jax.ShapeDtypeStruct((N, D_OUT), jnp.float32),
    )(aggp, deg_in, b2r)


def _pad_tiles(arr, n_tiles, padded_len, dump):
    per = E // n_tiles
    pad = padded_len - per
    a2 = arr.reshape(n_tiles, per)
    if dump:
        fill = N + (jnp.arange(pad, dtype=jnp.int32) % (NP - N))
        block = jnp.broadcast_to(fill[None, :], (n_tiles, pad))
    else:
        block = jnp.zeros((n_tiles, pad), dtype=jnp.int32)
    return jnp.concatenate([a2, block], axis=1).reshape(-1)


def kernel(in_feat, edge_index, W1, b1, W2, b2):
    src, dst = edge_index[0], edge_index[1]
    src_a = _pad_tiles(src, NW, EPTA, dump=False)
    dst_a = _pad_tiles(dst, NW, EPTA, dump=True)
    src_d = _pad_tiles(src, NS, EPTD, dump=True)
    dst_d = _pad_tiles(dst, NS, EPTD, dump=True)

    z16 = jnp.zeros((N, DEG_W), jnp.float32)
    ones = jnp.ones((CH, DEG_W), jnp.float32)
    z128 = jnp.zeros((N, D_H), jnp.float32)
    z64 = jnp.zeros((N, D_OUT), jnp.float32)

    deg_out, deg_in = _deg_kernel(src_d, dst_d, z16, ones)
    u = _matmul(in_feat, W1)
    h1 = _scale(u, deg_out)
    agg1 = _agg128(h1, src_a, dst_a, z128)
    h2 = _layer2_in(agg1, deg_in, deg_out, b1.reshape(1, D_H), W2)
    agg2 = _agg64(h2, src_a, dst_a, z64)
    return _final(agg2, deg_in, b2.reshape(1, D_OUT))


# speedup vs baseline: 1.3213x; 1.3023x over previous
import functools

import jax
import jax.numpy as jnp
from jax import lax
from jax.experimental import pallas as pl
from jax.experimental.pallas import tpu as pltpu
from jax.experimental.pallas import tpu_sc as plsc

N = 10000
E = 320000
D_IN = 128
D_H = 128
D_OUT = 64

NC = 2
NS = 16
NW = NC * NS
CH = 80

EPTA = E // NW
NCHA = EPTA // CH

EPTD = E // NS
NCHD = EPTD // CH

NP = N
WR = 632
WR_LAST = N - (NS - 1) * WR
DEG_W = 16

_MESH = plsc.VectorSubcoreMesh(
    core_axis_name="c", subcore_axis_name="s", num_cores=NC, num_subcores=NS
)
_PARAMS = pltpu.CompilerParams(use_tc_tiling_on_sc=False)


@functools.partial(
    pl.kernel,
    out_type=(
        jax.ShapeDtypeStruct((N, DEG_W), jnp.float32),
        jax.ShapeDtypeStruct((N, DEG_W), jnp.float32),
    ),
    mesh=_MESH,
    compiler_params=_PARAMS,
    scratch_types=[
        pltpu.VMEM_SHARED((NP, DEG_W), jnp.float32),
        pltpu.VMEM((CH,), jnp.int32),
        pltpu.VMEM((CH,), jnp.int32),
        pltpu.VMEM((CH, DEG_W), jnp.float32),
        pltpu.SemaphoreType.DMA,
        pltpu.SemaphoreType.DMA,
    ],
)
def _deg_kernel(srcp, dstp, z16, ones, dout, din, acc, i0, i1, ones_v, semi0, semi1):
    c = lax.axis_index("c")
    s = lax.axis_index("s")
    ebase = s * EPTD
    pltpu.sync_copy(ones, ones_v)

    ibuf = (i0, i1)
    semi = (semi0, semi1)

    @pl.when(s == 0)
    def _():
        pltpu.sync_copy(z16, acc.at[pl.ds(0, N)])

    def run(arr, out):
        def load_idx(j, p):
            pltpu.async_copy(arr.at[pl.ds(ebase + j * CH, CH)], ibuf[p], semi[p])

        def wait_idx(j, p):
            pltpu.make_async_copy(
                arr.at[pl.ds(ebase + j * CH, CH)], ibuf[p], semi[p]
            ).wait()

        load_idx(0, 0)
        load_idx(1, 1)
        plsc.subcore_barrier()

        def body(j, _):
            for p in (0, 1):
                @pl.when(j % 2 == p)
                def _():
                    wait_idx(j, p)
                    pltpu.sync_copy(ones_v, acc.at[ibuf[p]], add=True)

                    @pl.when(j + 2 < NCHD)
                    def _():
                        load_idx(j + 2, p)

            return ()

        lax.fori_loop(0, NCHD, body, ())
        plsc.subcore_barrier()

        @pl.when(s < NS - 1)
        def _():
            sl = pl.ds(s * WR, WR)
            pltpu.sync_copy(acc.at[sl], out.at[sl])

        @pl.when(s == NS - 1)
        def _():
            sl = pl.ds(s * WR, WR_LAST)
            pltpu.sync_copy(acc.at[sl], out.at[sl])

    @pl.when(c == 0)
    def _():
        run(srcp, dout)

    @pl.when(c == 1)
    def _():
        run(dstp, din)


def _make_agg(D):
    @functools.partial(
        pl.kernel,
        out_type=jax.ShapeDtypeStruct((NC, N, D), jnp.float32),
        mesh=_MESH,
        compiler_params=_PARAMS,
        scratch_types=[
            pltpu.VMEM_SHARED((NP, D), jnp.float32),
            pltpu.VMEM((CH,), jnp.int32),
            pltpu.VMEM((CH,), jnp.int32),
            pltpu.VMEM((CH,), jnp.int32),
            pltpu.VMEM((CH,), jnp.int32),
            pltpu.VMEM((CH, D), jnp.float32),
            pltpu.VMEM((CH, D), jnp.float32),
            pltpu.SemaphoreType.DMA,
            pltpu.SemaphoreType.DMA,
            pltpu.SemaphoreType.DMA,
            pltpu.SemaphoreType.DMA,
        ],
    )
    def agg(h, srcp, dstp, zd, out, acc,
            is0, is1, id0, id1, rows0, rows1, semi0, semi1, semg0, semg1):
        c = lax.axis_index("c")
        s = lax.axis_index("s")
        wid = c * NS + s
        ebase = wid * EPTA

        isbuf = (is0, is1)
        idbuf = (id0, id1)
        rows = (rows0, rows1)
        semi = (semi0, semi1)
        semg = (semg0, semg1)

        def load_idx(j, p):
            pltpu.async_copy(srcp.at[pl.ds(ebase + j * CH, CH)], isbuf[p], semi[p])
            pltpu.async_copy(dstp.at[pl.ds(ebase + j * CH, CH)], idbuf[p], semi[p])

        def wait_idx(j, p):
            pltpu.make_async_copy(
                srcp.at[pl.ds(ebase + j * CH, CH)], isbuf[p], semi[p]
            ).wait()
            pltpu.make_async_copy(
                dstp.at[pl.ds(ebase + j * CH, CH)], idbuf[p], semi[p]
            ).wait()

        def gather(p):
            pltpu.async_copy(h.at[isbuf[p]], rows[p], semg[p])

        def wait_gather(p):
            pltpu.make_async_copy(h.at[isbuf[p]], rows[p], semg[p]).wait()

        @pl.when(s == 0)
        def _():
            pltpu.sync_copy(zd, acc.at[pl.ds(0, N)])

        load_idx(0, 0)
        load_idx(1, 1)
        plsc.subcore_barrier()
        wait_idx(0, 0)
        gather(0)

        def body(j, _):
            for p in (0, 1):
                @pl.when(j % 2 == p)
                def _():
                    q = 1 - p

                    @pl.when(j + 1 < NCHA)
                    def _():
                        wait_idx(j + 1, q)
                        gather(q)

                    wait_gather(p)
                    pltpu.sync_copy(rows[p], acc.at[idbuf[p]], add=True)

                    @pl.when(j + 2 < NCHA)
                    def _():
                        load_idx(j + 2, p)

            return ()

        lax.fori_loop(0, NCHA, body, ())
        plsc.subcore_barrier()

        @pl.when(s < NS - 1)
        def _():
            sl = pl.ds(s * WR, WR)
            pltpu.sync_copy(acc.at[sl], out.at[c, sl])

        @pl.when(s == NS - 1)
        def _():
            sl = pl.ds(s * WR, WR_LAST)
            pltpu.sync_copy(acc.at[sl], out.at[c, sl])

    return agg


_agg128 = _make_agg(D_H)
_agg64 = _make_agg(D_OUT)


_BLK = 1000


def _norm_from(dp):
    return lax.rsqrt(jnp.maximum(dp[:, 0], 1.0))


def _matmul(x, w):
    m, k = x.shape
    d = w.shape[1]

    def body(x_ref, w_ref, o_ref):
        o_ref[...] = jnp.dot(x_ref[...], w_ref[...], preferred_element_type=jnp.float32)

    return pl.pallas_call(
        body,
        grid=(m // _BLK,),
        in_specs=[
            pl.BlockSpec((_BLK, k), lambda i: (i, 0)),
            pl.BlockSpec((k, d), lambda i: (0, 0)),
        ],
        out_specs=pl.BlockSpec((_BLK, d), lambda i: (i, 0)),
        out_shape=jax.ShapeDtypeStruct((m, d), jnp.float32),
    )(x, w)


def _scale(u, deg_out):
    m, d = u.shape

    def body(u_ref, dp_ref, o_ref):
        o_ref[...] = u_ref[...] * _norm_from(dp_ref[...])[:, None]

    return pl.pallas_call(
        body,
        grid=(m // _BLK,),
        in_specs=[
            pl.BlockSpec((_BLK, d), lambda i: (i, 0)),
            pl.BlockSpec((_BLK, DEG_W), lambda i: (i, 0)),
        ],
        out_specs=pl.BlockSpec((_BLK, d), lambda i: (i, 0)),
        out_shape=jax.ShapeDtypeStruct((m, d), jnp.float32),
    )(u, deg_out)


def _layer2_in(aggp, deg_in, deg_out, b1r, w2):
    d = w2.shape[1]

    def body(ap_ref, di_ref, do_ref, b_ref, w_ref, o_ref):
        ap = ap_ref[...]
        agg = ap[0] + ap[1]
        z = agg * _norm_from(di_ref[...])[:, None] + b_ref[...]
        z = jnp.maximum(z, 0.0)
        z = z * _norm_from(do_ref[...])[:, None]
        o_ref[...] = jnp.dot(z, w_ref[...], preferred_element_type=jnp.float32)

    return pl.pallas_call(
        body,
        grid=(N // _BLK,),
        in_specs=[
            pl.BlockSpec((NC, _BLK, D_H), lambda i: (0, i, 0)),
            pl.BlockSpec((_BLK, DEG_W), lambda i: (i, 0)),
            pl.BlockSpec((_BLK, DEG_W), lambda i: (i, 0)),
            pl.BlockSpec((1, D_H), lambda i: (0, 0)),
            pl.BlockSpec((D_H, d), lambda i: (0, 0)),
        ],
        out_specs=pl.BlockSpec((_BLK, d), lambda i: (i, 0)),
        out_shape=jax.ShapeDtypeStruct((N, d), jnp.float32),
    )(aggp, deg_in, deg_out, b1r, w2)


def _final(aggp, deg_in, b2r):
    def body(ap_ref, di_ref, b_ref, o_ref):
        ap = ap_ref[...]
        agg = ap[0] + ap[1]
        o_ref[...] = agg * _norm_from(di_ref[...])[:, None] + b_ref[...]

    return pl.pallas_call(
        body,
        grid=(N // _BLK,),
        in_specs=[
            pl.BlockSpec((NC, _BLK, D_OUT), lambda i: (0, i, 0)),
            pl.BlockSpec((_BLK, DEG_W), lambda i: (i, 0)),
            pl.BlockSpec((1, D_OUT), lambda i: (0, 0)),
        ],
        out_specs=pl.BlockSpec((_BLK, D_OUT), lambda i: (i, 0)),
        out_shape=jax.ShapeDtypeStruct((N, D_OUT), jnp.float32),
    )(aggp, deg_in, b2r)


def kernel(in_feat, edge_index, W1, b1, W2, b2):
    src, dst = edge_index[0], edge_index[1]
    src_a, dst_a = src, dst
    src_d, dst_d = src, dst

    z16 = jnp.zeros((N, DEG_W), jnp.float32)
    ones = jnp.ones((CH, DEG_W), jnp.float32)
    z128 = jnp.zeros((N, D_H), jnp.float32)
    z64 = jnp.zeros((N, D_OUT), jnp.float32)

    deg_out, deg_in = _deg_kernel(src_d, dst_d, z16, ones)
    u = _matmul(in_feat, W1)
    h1 = _scale(u, deg_out)
    agg1 = _agg128(h1, src_a, dst_a, z128)
    h2 = _layer2_in(agg1, deg_in, deg_out, b1.reshape(1, D_H), W2)
    agg2 = _agg64(h2, src_a, dst_a, z64)
    return _final(agg2, deg_in, b2.reshape(1, D_OUT))


# speedup vs baseline: 1.6464x; 1.2461x over previous
import functools

import jax
import jax.numpy as jnp
from jax import lax
from jax.experimental import pallas as pl
from jax.experimental.pallas import tpu as pltpu
from jax.experimental.pallas import tpu_sc as plsc

N = 10000
E = 320000
D_IN = 128
D_H = 128
D_OUT = 64

NC = 2
NS = 16
NW = NC * NS
CH = 80

EPTA = E // NW
NCHA = EPTA // CH

EPTD = E // NS
NCHD = EPTD // CH

NP = N
WR = 632
WR_LAST = N - (NS - 1) * WR
DEG_W = 16

_MESH = plsc.VectorSubcoreMesh(
    core_axis_name="c", subcore_axis_name="s", num_cores=NC, num_subcores=NS
)
_PARAMS = pltpu.CompilerParams(use_tc_tiling_on_sc=False)


@functools.partial(
    pl.kernel,
    out_type=(
        jax.ShapeDtypeStruct((N, DEG_W), jnp.float32),
        jax.ShapeDtypeStruct((N, DEG_W), jnp.float32),
    ),
    mesh=_MESH,
    compiler_params=_PARAMS,
    scratch_types=[
        pltpu.VMEM_SHARED((NP, DEG_W), jnp.float32),
        pltpu.VMEM((CH,), jnp.int32),
        pltpu.VMEM((CH,), jnp.int32),
        pltpu.VMEM((CH,), jnp.int32),
        pltpu.VMEM((CH,), jnp.int32),
        pltpu.VMEM((CH, DEG_W), jnp.float32),
        pltpu.SemaphoreType.DMA,
        pltpu.SemaphoreType.DMA,
        pltpu.SemaphoreType.DMA,
        pltpu.SemaphoreType.DMA,
        pltpu.SemaphoreType.DMA,
        pltpu.SemaphoreType.DMA,
        pltpu.SemaphoreType.DMA,
        pltpu.SemaphoreType.DMA,
    ],
)
def _deg_kernel(srcp, dstp, z16, ones, dout, din, acc,
                i0, i1, i2, i3, ones_v,
                semi0, semi1, semi2, semi3, semc0, semc1, semc2, semc3):
    c = lax.axis_index("c")
    s = lax.axis_index("s")
    ebase = s * EPTD
    pltpu.sync_copy(ones, ones_v)

    ibuf = (i0, i1, i2, i3)
    semi = (semi0, semi1, semi2, semi3)
    semc = (semc0, semc1, semc2, semc3)

    @pl.when(s == 0)
    def _():
        pltpu.sync_copy(z16, acc.at[pl.ds(0, N)])

    def run(arr, out):
        def load_idx(j, p):
            pltpu.async_copy(arr.at[pl.ds(ebase + j * CH, CH)], ibuf[p], semi[p])

        def wait_idx(j, p):
            pltpu.make_async_copy(
                arr.at[pl.ds(ebase + j * CH, CH)], ibuf[p], semi[p]
            ).wait()

        def scatter(p):
            pltpu.async_copy(ones_v, acc.at[ibuf[p]], semc[p], add=True)

        def wait_scatter(p):
            pltpu.make_async_copy(ones_v, acc.at[ibuf[p]], semc[p]).wait()

        load_idx(0, 0)
        load_idx(1, 1)
        plsc.subcore_barrier()

        def body(j, _):
            for p in range(4):
                @pl.when(j % 4 == p)
                def _():
                    wait_idx(j, p)
                    scatter(p)

                    @pl.when(j >= 2)
                    def _():
                        wait_scatter((p + 2) % 4)

                    @pl.when(j + 2 < NCHD)
                    def _():
                        load_idx(j + 2, (p + 2) % 4)

            return ()

        lax.fori_loop(0, NCHD, body, ())
        wait_scatter((NCHD - 2) % 4)
        wait_scatter((NCHD - 1) % 4)
        plsc.subcore_barrier()

        @pl.when(s < NS - 1)
        def _():
            sl = pl.ds(s * WR, WR)
            pltpu.sync_copy(acc.at[sl], out.at[sl])

        @pl.when(s == NS - 1)
        def _():
            sl = pl.ds(s * WR, WR_LAST)
            pltpu.sync_copy(acc.at[sl], out.at[sl])

    @pl.when(c == 0)
    def _():
        run(srcp, dout)

    @pl.when(c == 1)
    def _():
        run(dstp, din)


def _make_agg(D):
    @functools.partial(
        pl.kernel,
        out_type=jax.ShapeDtypeStruct((NC, N, D), jnp.float32),
        mesh=_MESH,
        compiler_params=_PARAMS,
        scratch_types=[
            pltpu.VMEM_SHARED((NP, D), jnp.float32),
            [pltpu.VMEM((CH,), jnp.int32)] * 8,
            [pltpu.VMEM((CH,), jnp.int32)] * 8,
            [pltpu.VMEM((CH, D), jnp.float32)] * 4,
            [pltpu.SemaphoreType.DMA] * 8,
            [pltpu.SemaphoreType.DMA] * 4,
            [pltpu.SemaphoreType.DMA] * 4,
        ],
    )
    def agg(h, srcp, dstp, zd, out, acc, isbuf, idbuf, rows, semi, semg, semc):
        c = lax.axis_index("c")
        s = lax.axis_index("s")
        wid = c * NS + s
        ebase = wid * EPTA

        def load_idx(j, p8):
            pltpu.async_copy(srcp.at[pl.ds(ebase + j * CH, CH)], isbuf[p8], semi[p8])
            pltpu.async_copy(dstp.at[pl.ds(ebase + j * CH, CH)], idbuf[p8], semi[p8])

        def wait_idx(j, p8):
            pltpu.make_async_copy(
                srcp.at[pl.ds(ebase + j * CH, CH)], isbuf[p8], semi[p8]
            ).wait()
            pltpu.make_async_copy(
                dstp.at[pl.ds(ebase + j * CH, CH)], idbuf[p8], semi[p8]
            ).wait()

        def gather(p8):
            pltpu.async_copy(h.at[isbuf[p8]], rows[p8 % 4], semg[p8 % 4])

        def wait_gather(p8):
            pltpu.make_async_copy(h.at[isbuf[p8]], rows[p8 % 4], semg[p8 % 4]).wait()

        def scatter(p8):
            pltpu.async_copy(rows[p8 % 4], acc.at[idbuf[p8]], semc[p8 % 4], add=True)

        def wait_scatter(p8):
            pltpu.make_async_copy(rows[p8 % 4], acc.at[idbuf[p8]], semc[p8 % 4]).wait()

        @pl.when(s == 0)
        def _():
            pltpu.sync_copy(zd, acc.at[pl.ds(0, N)])

        for j in range(4):
            load_idx(j, j)
        plsc.subcore_barrier()
        wait_idx(0, 0)
        gather(0)
        wait_idx(1, 1)
        gather(1)

        def body(j, _):
            for p in range(8):
                @pl.when(j % 8 == p)
                def _():
                    wait_gather(p)
                    scatter(p)
                    q = (p + 2) % 8

                    @pl.when(j + 2 < NCHA)
                    def _():
                        @pl.when(j >= 2)
                        def _():
                            wait_scatter((p + 6) % 8)

                        wait_idx(j + 2, q)
                        gather(q)

                    @pl.when(j + 4 < NCHA)
                    def _():
                        load_idx(j + 4, (p + 4) % 8)

            return ()

        lax.fori_loop(0, NCHA, body, ())
        for k in range(NCHA - 4, NCHA):
            wait_scatter(k % 8)
        plsc.subcore_barrier()

        @pl.when(s < NS - 1)
        def _():
            sl = pl.ds(s * WR, WR)
            pltpu.sync_copy(acc.at[sl], out.at[c, sl])

        @pl.when(s == NS - 1)
        def _():
            sl = pl.ds(s * WR, WR_LAST)
            pltpu.sync_copy(acc.at[sl], out.at[c, sl])

    return agg


_agg128 = _make_agg(D_H)
_agg64 = _make_agg(D_OUT)


_BLK = 1000


def _norm_from(dp):
    return lax.rsqrt(jnp.maximum(dp[:, 0], 1.0))


def _mm_scale(x, w, deg_out):
    m, k = x.shape
    d = w.shape[1]

    def body(x_ref, w_ref, dp_ref, o_ref):
        xw = jnp.dot(x_ref[...], w_ref[...], preferred_element_type=jnp.float32)
        o_ref[...] = xw * _norm_from(dp_ref[...])[:, None]

    return pl.pallas_call(
        body,
        grid=(m // _BLK,),
        in_specs=[
            pl.BlockSpec((_BLK, k), lambda i: (i, 0)),
            pl.BlockSpec((k, d), lambda i: (0, 0)),
            pl.BlockSpec((_BLK, DEG_W), lambda i: (i, 0)),
        ],
        out_specs=pl.BlockSpec((_BLK, d), lambda i: (i, 0)),
        out_shape=jax.ShapeDtypeStruct((m, d), jnp.float32),
    )(x, w, deg_out)


def _layer2_in(aggp, deg_in, deg_out, b1r, w2):
    d = w2.shape[1]

    def body(ap_ref, di_ref, do_ref, b_ref, w_ref, o_ref):
        ap = ap_ref[...]
        agg = ap[0] + ap[1]
        z = agg * _norm_from(di_ref[...])[:, None] + b_ref[...]
        z = jnp.maximum(z, 0.0)
        z = z * _norm_from(do_ref[...])[:, None]
        o_ref[...] = jnp.dot(z, w_ref[...], preferred_element_type=jnp.float32)

    return pl.pallas_call(
        body,
        grid=(N // _BLK,),
        in_specs=[
            pl.BlockSpec((NC, _BLK, D_H), lambda i: (0, i, 0)),
            pl.BlockSpec((_BLK, DEG_W), lambda i: (i, 0)),
            pl.BlockSpec((_BLK, DEG_W), lambda i: (i, 0)),
            pl.BlockSpec((1, D_H), lambda i: (0, 0)),
            pl.BlockSpec((D_H, d), lambda i: (0, 0)),
        ],
        out_specs=pl.BlockSpec((_BLK, d), lambda i: (i, 0)),
        out_shape=jax.ShapeDtypeStruct((N, d), jnp.float32),
    )(aggp, deg_in, deg_out, b1r, w2)


def _final(aggp, deg_in, b2r):
    def body(ap_ref, di_ref, b_ref, o_ref):
        ap = ap_ref[...]
        agg = ap[0] + ap[1]
        o_ref[...] = agg * _norm_from(di_ref[...])[:, None] + b_ref[...]

    return pl.pallas_call(
        body,
        grid=(N // _BLK,),
        in_specs=[
            pl.BlockSpec((NC, _BLK, D_OUT), lambda i: (0, i, 0)),
            pl.BlockSpec((_BLK, DEG_W), lambda i: (i, 0)),
            pl.BlockSpec((1, D_OUT), lambda i: (0, 0)),
        ],
        out_specs=pl.BlockSpec((_BLK, D_OUT), lambda i: (i, 0)),
        out_shape=jax.ShapeDtypeStruct((N, D_OUT), jnp.float32),
    )(aggp, deg_in, b2r)


def kernel(in_feat, edge_index, W1, b1, W2, b2):
    src, dst = edge_index[0], edge_index[1]
    src_a, dst_a = src, dst
    src_d, dst_d = src, dst

    z16 = jnp.zeros((N, DEG_W), jnp.float32)
    ones = jnp.ones((CH, DEG_W), jnp.float32)
    z128 = jnp.zeros((N, D_H), jnp.float32)
    z64 = jnp.zeros((N, D_OUT), jnp.float32)

    deg_out, deg_in = _deg_kernel(src_d, dst_d, z16, ones)
    h1 = _mm_scale(in_feat, W1, deg_out)
    agg1 = _agg128(h1, src_a, dst_a, z128)
    h2 = _layer2_in(agg1, deg_in, deg_out, b1.reshape(1, D_H), W2)
    agg2 = _agg64(h2, src_a, dst_a, z64)
    return _final(agg2, deg_in, b2.reshape(1, D_OUT))
